# Initial kernel scaffold; baseline (speedup 1.0000x reference)
import jax
import jax.numpy as jnp
from jax.experimental import pallas as pl


def kernel(x, edge_index, params):
    raise NotImplementedError("write your pallas kernel here")


# speedup vs baseline: 6.6549x; 6.6549x over previous
import functools

import jax
import jax.numpy as jnp
from jax import lax
from jax.experimental import pallas as pl
from jax.experimental.pallas import tpu as pltpu
from jax.experimental.pallas import tpu_sc as plsc

_N = 10000
_E = 320000
_DIN = 128
_H = 64

_NC = 2
_NS = 16
_NW = _NC * _NS
_EPW = _E // _NW
_C = 80
_NCH = _EPW // _C
_NPAD = 10240
_RPT = _NPAD // _NS

_f32 = jnp.float32
_PREC = jax.lax.Precision.HIGHEST


def _agg_body(h_hbm, src_hbm, dst_hbm, zeros_hbm, out_hbm,
              src_v, dst_v, buf, sem, accum):
    cid = lax.axis_index("c")
    sid = lax.axis_index("s")
    wid = sid * _NC + cid

    r0 = sid * _RPT
    pltpu.sync_copy(zeros_hbm.at[pl.ds(r0, _RPT)], accum.at[pl.ds(r0, _RPT)])
    pltpu.sync_copy(src_hbm.at[wid], src_v)
    pltpu.sync_copy(dst_hbm.at[wid], dst_v)
    plsc.subcore_barrier()

    def chunk(j, carry):
        pltpu.async_copy(h_hbm.at[src_v.at[j]], buf, sem).wait()
        pltpu.sync_copy(buf, accum.at[dst_v.at[j]], add=True)
        return carry

    lax.fori_loop(0, _NCH, chunk, 0)
    plsc.subcore_barrier()
    pltpu.sync_copy(accum.at[pl.ds(r0, _RPT)], out_hbm.at[cid, pl.ds(r0, _RPT)])


@functools.partial(jax.jit, static_argnums=3)
def _agg(h, src_r, dst_r, d):
    zeros = jnp.zeros((_NPAD, d), _f32)
    mesh = plsc.VectorSubcoreMesh(
        core_axis_name="c", subcore_axis_name="s", num_cores=_NC,
        num_subcores=_NS)
    return pl.kernel(
        _agg_body,
        out_type=jax.ShapeDtypeStruct((_NC, _NPAD, d), _f32),
        mesh=mesh,
        compiler_params=pltpu.CompilerParams(use_tc_tiling_on_sc=False),
        scratch_types=[
            pltpu.VMEM((_NCH, _C), jnp.int32),
            pltpu.VMEM((_NCH, _C), jnp.int32),
            pltpu.VMEM((_C, d), _f32),
            pltpu.SemaphoreType.DMA,
            pltpu.VMEM_SHARED((_NPAD, d), _f32),
        ],
    )(h, src_r, dst_r, zeros)


def _bn(t, gamma, beta):
    m = jnp.mean(t, axis=0, keepdims=True)
    v = jnp.mean((t - m) ** 2, axis=0, keepdims=True)
    return gamma * (t - m) / jnp.sqrt(v + 1e-5) + beta


def _gin_mlp(h_ref, p_ref, eps_ref, w1_ref, b1_ref, g1_ref, bb1_ref, w2_ref,
             b2_ref, go_ref, bo_ref):
    t = (1.0 + eps_ref[...]) * h_ref[...] + p_ref[0, :_N, :] + p_ref[1, :_N, :]
    t = jnp.dot(t, w1_ref[...], preferred_element_type=_f32,
                precision=_PREC) + b1_ref[...]
    t = jnp.maximum(_bn(t, g1_ref[...], bb1_ref[...]), 0.0)
    u = jnp.dot(t, w2_ref[...], preferred_element_type=_f32,
                precision=_PREC) + b2_ref[...]
    return jnp.maximum(_bn(u, go_ref[...], bo_ref[...]), 0.0)


def _score(h, wp_ref, bp_ref):
    return jnp.dot(jnp.sum(h, axis=0, keepdims=True), wp_ref[...],
                   preferred_element_type=_f32, precision=_PREC) + bp_ref[...]


def _layer0_body(h_ref, p_ref, eps_ref, w1_ref, b1_ref, g1_ref, bb1_ref,
                 w2_ref, b2_ref, go_ref, bo_ref, wp0_ref, bp0_ref, wp_ref,
                 bp_ref, hn_ref, s_ref):
    h = _gin_mlp(h_ref, p_ref, eps_ref, w1_ref, b1_ref, g1_ref, bb1_ref,
                 w2_ref, b2_ref, go_ref, bo_ref)
    hn_ref[...] = h
    s_ref[...] = (_score(h_ref[...], wp0_ref, bp0_ref)
                  + _score(h, wp_ref, bp_ref))


def _mid_body(h_ref, p_ref, eps_ref, w1_ref, b1_ref, g1_ref, bb1_ref, w2_ref,
              b2_ref, go_ref, bo_ref, wp_ref, bp_ref, hn_ref, s_ref):
    h = _gin_mlp(h_ref, p_ref, eps_ref, w1_ref, b1_ref, g1_ref, bb1_ref,
                 w2_ref, b2_ref, go_ref, bo_ref)
    hn_ref[...] = h
    s_ref[...] = _score(h, wp_ref, bp_ref)


def _last_body(h_ref, p_ref, eps_ref, w1_ref, b1_ref, g1_ref, bb1_ref,
               w2_ref, b2_ref, go_ref, bo_ref, wp_ref, bp_ref, s_ref):
    h = _gin_mlp(h_ref, p_ref, eps_ref, w1_ref, b1_ref, g1_ref, bb1_ref,
                 w2_ref, b2_ref, go_ref, bo_ref)
    s_ref[...] = _score(h, wp_ref, bp_ref)


@jax.jit
def _tc_layer0(*args):
    return pl.pallas_call(
        _layer0_body,
        out_shape=(jax.ShapeDtypeStruct((_N, _H), _f32),
                   jax.ShapeDtypeStruct((1, 1), _f32)),
    )(*args)


@jax.jit
def _tc_mid(*args):
    return pl.pallas_call(
        _mid_body,
        out_shape=(jax.ShapeDtypeStruct((_N, _H), _f32),
                   jax.ShapeDtypeStruct((1, 1), _f32)),
    )(*args)


@jax.jit
def _tc_last(*args):
    return pl.pallas_call(
        _last_body,
        out_shape=jax.ShapeDtypeStruct((1, 1), _f32),
    )(*args)


def kernel(x, edge_index, params):
    src_r = edge_index[0].reshape(_NW, _NCH, _C)
    dst_r = edge_index[1].reshape(_NW, _NCH, _C)

    def row(v):
        return v.reshape(1, -1)

    h = x
    out = None
    for l in range(4):
        p = _agg(h, src_r, dst_r, _DIN if l == 0 else _H)
        layer = (p, params[f"eps_{l}"].reshape(1, 1), params[f"W1_{l}"],
                 row(params[f"b1_{l}"]), row(params[f"g1_{l}"]),
                 row(params[f"bb1_{l}"]), params[f"W2_{l}"],
                 row(params[f"b2_{l}"]), row(params[f"go_{l}"]),
                 row(params[f"bo_{l}"]))
        wp = params[f"Wp_{l + 1}"]
        bp = params[f"bp_{l + 1}"].reshape(1, 1)
        if l == 0:
            h, s = _tc_layer0(h, *layer, params["Wp_0"],
                              params["bp_0"].reshape(1, 1), wp, bp)
        elif l < 3:
            h, s = _tc_mid(h, *layer, wp, bp)
        else:
            s = _tc_last(h, *layer, wp, bp)
        out = s if out is None else out + s
    return out
